# Initial kernel scaffold; baseline (speedup 1.0000x reference)
import jax
import jax.numpy as jnp
from jax.experimental import pallas as pl


def kernel(s_cat, s_cont, k_cat, k_cont, o_cat, o_cont, target, s_cat_tables, k_cat_tables, o_cat_tables, s_cont_emb, s_cont_bias, k_cont_emb, k_cont_bias, o_cont_emb, o_cont_bias, tgt_emb, tgt_bias):
    raise NotImplementedError("write your pallas kernel here")


# speedup vs baseline: 1.1338x; 1.1338x over previous
import functools

import jax
import jax.numpy as jnp
from jax import lax
from jax.experimental import pallas as pl
from jax.experimental.pallas import tpu as pltpu
from jax.experimental.pallas import tpu_sc as plsc

_B = 4096
_T = 200
_H = 64
_BT = _B * _T
_NW = 32
_C = 128

_PER = _BT // _NW
_SPER = _B // _NW


def _sc_gather_body(k0i, k1i, oi, s0i, s1i, s2i,
                    kt0, kt1, ot, st0, st1, st2,
                    tk0, tk1, to, ts0, ts1, ts2,
                    idx_v, rows_v, sem):
    wid = lax.axis_index("s") * 2 + lax.axis_index("c")

    def stream(idx_hbm, tab_hbm, out_hbm, base, nrows):
        @pl.loop(0, nrows // _C)
        def chunk(i):
            off = base + i * _C
            pltpu.sync_copy(idx_hbm.at[pl.ds(off, _C)], idx_v)
            pltpu.async_copy(tab_hbm.at[idx_v], rows_v, sem).wait()
            pltpu.sync_copy(rows_v, out_hbm.at[pl.ds(off, _C)])

    base = wid * _PER
    stream(k0i, kt0, tk0, base, _PER)
    stream(k1i, kt1, tk1, base, _PER)
    stream(oi, ot, to, base, _PER)
    sbase = wid * _SPER
    stream(s0i, st0, ts0, sbase, _SPER)
    stream(s1i, st1, ts1, sbase, _SPER)
    stream(s2i, st2, ts2, sbase, _SPER)


def _sc_gather(k0i, k1i, oi, s0i, s1i, s2i, kt0, kt1, ot, st0, st1, st2):
    mesh = plsc.VectorSubcoreMesh(core_axis_name="c", subcore_axis_name="s")
    row = lambda n: jax.ShapeDtypeStruct((n, _H), jnp.float32)
    f = pl.kernel(
        _sc_gather_body,
        out_type=(row(_BT), row(_BT), row(_BT), row(_B), row(_B), row(_B)),
        mesh=mesh,
        scratch_types=[
            pltpu.VMEM((_C,), jnp.int32),
            pltpu.VMEM((_C, _H), jnp.float32),
            pltpu.SemaphoreType.DMA,
        ],
        compiler_params=pltpu.CompilerParams(use_tc_tiling_on_sc=False),
    )
    return f(k0i, k1i, oi, s0i, s1i, s2i, kt0, kt1, ot, st0, st1, st2)


_NP = 512


def _asm_big_body(tk0, tk1, kc, ke, kb, to, oc, oe, ob, tg, te, tb,
                  outk, outo, outt):
    outk[:, 0, :] = tk0[...]
    outk[:, 1, :] = tk1[...]
    outk[:, 2:, :] = kc[...][:, :, None] * ke[...][None] + kb[...][None]
    outo[:, 0, :] = to[...]
    outo[:, 1:, :] = oc[...][:, :, None] * oe[...][None] + ob[...][None]
    outt[...] = tg[...][:, :, None] * te[...][None] + tb[...][None]


def _asm_big(tk0, tk1, kc, ke, kb, to, oc, oe, ob, tg, te, tb):
    n = _NP
    grid = (_BT // n,)
    blk_row = pl.BlockSpec((n, _H), lambda i: (i, 0))
    blk_full = lambda r, c: pl.BlockSpec((r, c), lambda i: (0, 0))
    return pl.pallas_call(
        _asm_big_body,
        grid=grid,
        in_specs=[
            blk_row,
            blk_row,
            pl.BlockSpec((n, 8), lambda i: (i, 0)),
            blk_full(8, _H), blk_full(8, _H),
            blk_row,
            pl.BlockSpec((n, 8), lambda i: (i, 0)),
            blk_full(8, _H), blk_full(8, _H),
            pl.BlockSpec((n, 1), lambda i: (i, 0)),
            blk_full(1, _H), blk_full(1, _H),
        ],
        out_specs=[
            pl.BlockSpec((n, 10, _H), lambda i: (i, 0, 0)),
            pl.BlockSpec((n, 9, _H), lambda i: (i, 0, 0)),
            pl.BlockSpec((n, 1, _H), lambda i: (i, 0, 0)),
        ],
        out_shape=[
            jax.ShapeDtypeStruct((_BT, 10, _H), jnp.float32),
            jax.ShapeDtypeStruct((_BT, 9, _H), jnp.float32),
            jax.ShapeDtypeStruct((_BT, 1, _H), jnp.float32),
        ],
    )(tk0, tk1, kc, ke, kb, to, oc, oe, ob, tg, te, tb)


def _asm_s_body(ts0, ts1, ts2, sc, se, sb, outs):
    outs[:, 0, :] = ts0[...]
    outs[:, 1, :] = ts1[...]
    outs[:, 2, :] = ts2[...]
    outs[:, 3:, :] = sc[...][:, :, None] * se[...][None] + sb[...][None]


def _asm_s(ts0, ts1, ts2, sc, se, sb):
    n = _NP
    grid = (_B // n,)
    blk_row = pl.BlockSpec((n, _H), lambda i: (i, 0))
    return pl.pallas_call(
        _asm_s_body,
        grid=grid,
        in_specs=[
            blk_row, blk_row, blk_row,
            pl.BlockSpec((n, 4), lambda i: (i, 0)),
            pl.BlockSpec((4, _H), lambda i: (0, 0)),
            pl.BlockSpec((4, _H), lambda i: (0, 0)),
        ],
        out_specs=pl.BlockSpec((n, 7, _H), lambda i: (i, 0, 0)),
        out_shape=jax.ShapeDtypeStruct((_B, 7, _H), jnp.float32),
    )(ts0, ts1, ts2, sc, se, sb)


def kernel(s_cat, s_cont, k_cat, k_cont, o_cat, o_cont, target,
           s_cat_tables, k_cat_tables, o_cat_tables,
           s_cont_emb, s_cont_bias, k_cont_emb, k_cont_bias,
           o_cont_emb, o_cont_bias, tgt_emb, tgt_bias):
    k0i = k_cat[:, :, 0].reshape(_BT)
    k1i = k_cat[:, :, 1].reshape(_BT)
    oi = o_cat[:, :, 0].reshape(_BT)
    s0i = s_cat[:, 0, 0]
    s1i = s_cat[:, 0, 1]
    s2i = s_cat[:, 0, 2]

    tk0, tk1, to, ts0, ts1, ts2 = _sc_gather(
        k0i, k1i, oi, s0i, s1i, s2i,
        k_cat_tables[0], k_cat_tables[1], o_cat_tables[0],
        s_cat_tables[0], s_cat_tables[1], s_cat_tables[2])

    kc = k_cont.reshape(_BT, 8)
    oc = o_cont.reshape(_BT, 8)
    tg = target.reshape(_BT, 1)
    outk, outo, outt = _asm_big(tk0, tk1, kc, k_cont_emb, k_cont_bias,
                                to, oc, o_cont_emb, o_cont_bias,
                                tg, tgt_emb, tgt_bias)
    outs = _asm_s(ts0, ts1, ts2, s_cont[:, 0, :], s_cont_emb, s_cont_bias)

    return (outs,
            outk.reshape(_B, _T, 10, _H),
            outo.reshape(_B, _T, 9, _H),
            outt.reshape(_B, _T, 1, _H))
